# baseline (speedup 1.0000x reference)
import functools

import jax
import jax.numpy as jnp
from jax import lax
from jax.experimental import pallas as pl
from jax.experimental.pallas import tpu as pltpu
from jax.experimental.pallas import tpu_sc as plsc

_B = 4096
_GT, _GN, _GD = 13, 100000, 64
_CT, _CN, _CD = 13, 1000000, 32

_NC, _NS = 2, 16
_NW = _NC * _NS
_BPW = _B // _NW


def _sc_gather(gt2, gidx, ct2, cidx):
    mesh = plsc.VectorSubcoreMesh(
        core_axis_name="c", subcore_axis_name="s",
        num_cores=_NC, num_subcores=_NS)

    @functools.partial(
        pl.kernel,
        out_type=(jax.ShapeDtypeStruct((_GT, _B, _GD), jnp.float32),
                  jax.ShapeDtypeStruct((_CT, _B, _CD), jnp.float32)),
        mesh=mesh,
        compiler_params=pltpu.CompilerParams(use_tc_tiling_on_sc=False),
        scratch_types=(
            pltpu.VMEM((1, _GT * _BPW), jnp.int32),
            pltpu.VMEM((1, _CT * _BPW), jnp.int32),
            pltpu.VMEM((_BPW, _GD), jnp.float32),
            pltpu.VMEM((_BPW, _GD), jnp.float32),
            pltpu.VMEM((_BPW, _CD), jnp.float32),
            pltpu.VMEM((_BPW, _CD), jnp.float32),
            pltpu.SemaphoreType.DMA,
            pltpu.SemaphoreType.DMA,
        ),
    )
    def k(gt_hbm, gidx_hbm, ct_hbm, cidx_hbm, gout_hbm, cout_hbm,
          gi_v, ci_v, g0, g1, c0, c1, sem0, sem1):
        wid = lax.axis_index("s") * _NC + lax.axis_index("c")
        rbase = wid * _BPW
        pltpu.sync_copy(gidx_hbm.at[wid], gi_v)
        pltpu.sync_copy(cidx_hbm.at[wid], ci_v)
        sems = (sem0, sem1)
        for tab, idx_v, out, bufs, nt in (
                (gt_hbm, gi_v, gout_hbm, (g0, g1), _GT),
                (ct_hbm, ci_v, cout_hbm, (c0, c1), _CT)):
            copies = [None, None]
            for t in range(nt):
                p = t & 1
                if copies[p] is not None:
                    copies[p].wait()
                idx = idx_v.at[0, pl.ds(t * _BPW, _BPW)]
                pltpu.async_copy(tab.at[idx], bufs[p], sems[p]).wait()
                copies[p] = pltpu.async_copy(
                    bufs[p], out.at[t, pl.ds(rbase, _BPW)], sems[p])
            copies[0].wait()
            copies[1].wait()

    return k(gt2, gidx, ct2, cidx)


def _mlp_body(df, gp, cp, dw1t, db1, dw2t, db2,
              w1g, w1c, w1dt, ob1, ow2t, ob2, ow3t, ob3, ow4t, ob4,
              ow5t, ob5, out):
    dot = functools.partial(jnp.dot, preferred_element_type=jnp.float32)
    h = jnp.maximum(dot(df[...], dw1t[...]) + db1[...], 0.0)
    de = dot(h, dw2t[...]) + db2[...]
    o = dot(de, w1dt[...]) + ob1[...]
    gpv = gp[...]
    cpv = cp[...]
    w1gv = w1g[...]
    w1cv = w1c[...]
    for t in range(_GT):
        o = o + dot(gpv[t], w1gv[t])
    for t in range(_CT):
        o = o + dot(cpv[t], w1cv[t])
    o = jnp.maximum(o, 0.0)
    o = jnp.maximum(dot(o, ow2t[...]) + ob2[...], 0.0)
    o = jnp.maximum(dot(o, ow3t[...]) + ob3[...], 0.0)
    o = jnp.maximum(dot(o, ow4t[...]) + ob4[...], 0.0)
    out[...] = dot(o, ow5t[...]) + ob5[...]


def _tc_mlp(df, gp3, cp3, dw1t, db1, dw2t, db2,
            w1g, w1c, w1dt, ob1, ow2t, ob2, ow3t, ob3, ow4t, ob4,
            ow5t, ob5, block_b=512):
    grid = (_B // block_b,)

    def full_spec(a):
        return pl.BlockSpec(a.shape, lambda i: (0,) * a.ndim)

    weights = (dw1t, db1, dw2t, db2, w1g, w1c, w1dt, ob1,
               ow2t, ob2, ow3t, ob3, ow4t, ob4, ow5t, ob5)
    return pl.pallas_call(
        _mlp_body,
        grid=grid,
        in_specs=[pl.BlockSpec((block_b, df.shape[1]), lambda i: (i, 0)),
                  pl.BlockSpec((_GT, block_b, _GD), lambda i: (0, i, 0)),
                  pl.BlockSpec((_CT, block_b, _CD), lambda i: (0, i, 0))]
                 + [full_spec(w) for w in weights],
        out_specs=pl.BlockSpec((block_b, 1), lambda i: (i, 0)),
        out_shape=jax.ShapeDtypeStruct((_B, 1), jnp.float32),
    )(df, gp3, cp3, *weights)


def _worker_major(idx):
    return (idx.T.reshape(idx.shape[1], _NW, _BPW)
            .transpose(1, 0, 2).reshape(_NW, 1, -1))


def kernel(dense_features, gpu_sharded_sparse_features, cpu_sharded_sparse_features,
           gpu_tables, cpu_tables, dw1, db1, dw2, db2,
           ow1, ob1, ow2, ob2, ow3, ob3, ow4, ob4, ow5, ob5):
    gflat = (gpu_sharded_sparse_features.astype(jnp.int32)
             + jnp.arange(_GT, dtype=jnp.int32)[None, :] * _GN)
    cflat = (cpu_sharded_sparse_features.astype(jnp.int32)
             + jnp.arange(_CT, dtype=jnp.int32)[None, :] * _CN)
    gidx = _worker_major(gflat)
    cidx = _worker_major(cflat)

    gp3, cp3 = _sc_gather(
        gpu_tables.reshape(_GT * _GN, _GD), gidx,
        cpu_tables.reshape(_CT * _CN, _CD), cidx)

    ow1t = ow1.T
    g_cols = _GT * _GD
    c_cols = _CT * _CD
    w1g = ow1t[:g_cols].reshape(_GT, _GD, 512)
    w1c = ow1t[g_cols:g_cols + c_cols].reshape(_CT, _CD, 512)
    w1dt = ow1t[g_cols + c_cols:]

    return _tc_mlp(
        dense_features, gp3, cp3,
        dw1.T, db1[None, :], dw2.T, db2[None, :],
        w1g, w1c, w1dt, ob1[None, :],
        ow2.T, ob2[None, :], ow3.T, ob3[None, :], ow4.T, ob4[None, :],
        ow5.T, ob5[None, :])
